# baseline (speedup 1.0000x reference)
import functools

import jax
import jax.numpy as jnp
import numpy as np
from jax import lax
from jax.experimental import pallas as pl
from jax.experimental.pallas import tpu as pltpu
from jax.experimental.pallas import tpu_sc as plsc

VOCAB = 100000
D = 768
SEQ = 2048
BATCH = 4
ROWS = BATCH * SEQ
NC, NS, L = 2, 16, 16
NW = NC * NS
NSLC = 2
SROWS = ROWS // NSLC
RPW = SROWS // NW
CHUNK = 64
NCH = RPW // CHUNK
BLK = 512
EPS = 1e-6


def _positional_encoding(seq_len, d_model):
    pos = np.arange(seq_len, dtype=np.float64)[:, None]
    i = np.arange(0, d_model, 2, dtype=np.float64)
    div = np.exp(i * -(np.log(10000.0) / d_model))
    pe = np.zeros((seq_len, d_model), dtype=np.float64)
    pe[:, 0::2] = np.sin(pos * div)
    pe[:, 1::2] = np.cos(pos * div)
    return jnp.asarray(pe, dtype=jnp.float32)


def _sc_gather_body(seq_h, tok_h, out_h, idx_v, b0, b1, gs0, gs1, ws0, ws1):
    wid = lax.axis_index("s") * NC + lax.axis_index("c")
    base0 = wid * RPW
    pltpu.sync_copy(seq_h.at[pl.ds(base0, RPW)], idx_v)

    bufs = (b0, b1)
    gsems = (gs0, gs1)
    wsems = (ws0, ws1)

    def gather(c):
        return pltpu.async_copy(
            tok_h.at[idx_v.at[pl.ds(c * CHUNK, CHUNK)]],
            bufs[c % 2], gsems[c % 2])

    pend_g = gather(0)
    pend_w = [None, None]
    for c in range(NCH):
        pend_g.wait()
        if c + 1 < NCH:
            if pend_w[(c + 1) % 2] is not None:
                pend_w[(c + 1) % 2].wait()
            pend_g = gather(c + 1)
        pend_w[c % 2] = pltpu.async_copy(
            bufs[c % 2], out_h.at[pl.ds(base0 + c * CHUNK, CHUNK)],
            wsems[c % 2])
    for w in pend_w:
        if w is not None:
            w.wait()


def _sc_gather(seq_slice, token_table):
    mesh = plsc.VectorSubcoreMesh(core_axis_name="c", subcore_axis_name="s",
                                  num_cores=NC, num_subcores=NS)
    f = pl.kernel(
        _sc_gather_body,
        out_type=jax.ShapeDtypeStruct((SROWS, D), jnp.float32),
        mesh=mesh,
        scratch_types=[
            pltpu.VMEM((RPW,), jnp.int32),
            pltpu.VMEM((CHUNK, D), jnp.float32),
            pltpu.VMEM((CHUNK, D), jnp.float32),
            pltpu.SemaphoreType.DMA,
            pltpu.SemaphoreType.DMA,
            pltpu.SemaphoreType.DMA,
            pltpu.SemaphoreType.DMA,
        ],
    )
    return f(seq_slice, token_table)


def _tc_body(lbl_ref, x_ref, pe_ref, st_ref, gam_ref, bet_ref, o_ref):
    x = x_ref[...] + pe_ref[...]
    lbl = lbl_ref[...]
    for k in range(3):
        x = x + jnp.where(lbl == k, st_ref[k:k + 1, :],
                          jnp.float32(0.0))
    mu = jnp.mean(x, axis=-1, keepdims=True)
    var = jnp.mean((x - mu) * (x - mu), axis=-1, keepdims=True)
    o_ref[...] = ((x - mu) * lax.rsqrt(var + jnp.float32(EPS))
                  * gam_ref[...] + bet_ref[...])


def _tc_ln(slice_idx, lbl, x, pe, st, gamma, beta):
    nblk = SROWS // BLK
    pe_blocks = SEQ // BLK
    f = pl.pallas_call(
        _tc_body,
        grid=(nblk,),
        in_specs=[
            pl.BlockSpec((BLK, 1), lambda i: (i, 0)),
            pl.BlockSpec((BLK, D), lambda i: (i, 0)),
            pl.BlockSpec((BLK, D),
                         lambda i, s=slice_idx: (
                             (s * (SROWS // BLK) + i) % pe_blocks, 0)),
            pl.BlockSpec((3, D), lambda i: (0, 0)),
            pl.BlockSpec((1, D), lambda i: (0, 0)),
            pl.BlockSpec((1, D), lambda i: (0, 0)),
        ],
        out_specs=pl.BlockSpec((BLK, D), lambda i: (i, 0)),
        out_shape=jax.ShapeDtypeStruct((SROWS, D), jnp.float32),
    )
    return f(lbl, x, pe, st, gamma, beta)


@jax.jit
def _run(seq, seg, token_table, segtab, pe, gamma, beta):
    gathered = [
        _sc_gather(lax.dynamic_slice(seq, (s * SROWS,), (SROWS,)),
                   token_table)
        for s in range(NSLC)
    ]
    outs = [
        _tc_ln(s, lax.dynamic_slice(seg, (s * SROWS, 0), (SROWS, 1)),
               gathered[s], pe, segtab, gamma, beta)
        for s in range(NSLC)
    ]
    if NSLC == 1:
        return outs[0]
    return jnp.concatenate(outs, axis=0)


def kernel(sequence, segment_label, token_table, segment_table, gamma, beta):
    pe = _positional_encoding(SEQ, D)
    out = _run(sequence.reshape(-1), segment_label.reshape(-1, 1),
               token_table, segment_table,
               pe, gamma.reshape(1, D), beta.reshape(1, D))
    return out.reshape(BATCH, SEQ, D)


# speedup vs baseline: 1.2457x; 1.2457x over previous
import functools

import jax
import jax.numpy as jnp
import numpy as np
from jax import lax
from jax.experimental import pallas as pl
from jax.experimental.pallas import tpu as pltpu
from jax.experimental.pallas import tpu_sc as plsc

VOCAB = 100000
D = 768
SEQ = 2048
BATCH = 4
ROWS = BATCH * SEQ
NC, NS, L = 2, 16, 16
NW = NC * NS
NSLC = 1
SROWS = ROWS // NSLC
RPW = SROWS // NW
CHUNK = 64
NCH = RPW // CHUNK
BLK = 512
EPS = 1e-6


def _positional_encoding(seq_len, d_model):
    pos = np.arange(seq_len, dtype=np.float64)[:, None]
    i = np.arange(0, d_model, 2, dtype=np.float64)
    div = np.exp(i * -(np.log(10000.0) / d_model))
    pe = np.zeros((seq_len, d_model), dtype=np.float64)
    pe[:, 0::2] = np.sin(pos * div)
    pe[:, 1::2] = np.cos(pos * div)
    return jnp.asarray(pe, dtype=jnp.float32)


def _sc_gather_body(seq_h, tok_h, out_h, idx_v, b0, b1, gs0, gs1, ws0, ws1):
    wid = lax.axis_index("s") * NC + lax.axis_index("c")
    base0 = wid * RPW
    pltpu.sync_copy(seq_h.at[pl.ds(base0, RPW)], idx_v)

    bufs = (b0, b1)
    gsems = (gs0, gs1)
    wsems = (ws0, ws1)

    def gather(c):
        return pltpu.async_copy(
            tok_h.at[idx_v.at[pl.ds(c * CHUNK, CHUNK)]],
            bufs[c % 2], gsems[c % 2])

    pend_g = gather(0)
    pend_w = [None, None]
    for c in range(NCH):
        pend_g.wait()
        if c + 1 < NCH:
            if pend_w[(c + 1) % 2] is not None:
                pend_w[(c + 1) % 2].wait()
            pend_g = gather(c + 1)
        pend_w[c % 2] = pltpu.async_copy(
            bufs[c % 2], out_h.at[pl.ds(base0 + c * CHUNK, CHUNK)],
            wsems[c % 2])
    for w in pend_w:
        if w is not None:
            w.wait()


def _sc_gather(seq_slice, token_table):
    mesh = plsc.VectorSubcoreMesh(core_axis_name="c", subcore_axis_name="s",
                                  num_cores=NC, num_subcores=NS)
    f = pl.kernel(
        _sc_gather_body,
        out_type=jax.ShapeDtypeStruct((SROWS, D), jnp.float32),
        mesh=mesh,
        scratch_types=[
            pltpu.VMEM((RPW,), jnp.int32),
            pltpu.VMEM((CHUNK, D), jnp.float32),
            pltpu.VMEM((CHUNK, D), jnp.float32),
            pltpu.SemaphoreType.DMA,
            pltpu.SemaphoreType.DMA,
            pltpu.SemaphoreType.DMA,
            pltpu.SemaphoreType.DMA,
        ],
    )
    return f(seq_slice, token_table)


def _tc_body(lbl_ref, x_ref, pe_ref, st_ref, gam_ref, bet_ref, o_ref):
    x = x_ref[...] + pe_ref[...]
    lbl = lbl_ref[...]
    for k in range(3):
        x = x + jnp.where(lbl == k, st_ref[k:k + 1, :],
                          jnp.float32(0.0))
    mu = jnp.mean(x, axis=-1, keepdims=True)
    var = jnp.mean((x - mu) * (x - mu), axis=-1, keepdims=True)
    o_ref[...] = ((x - mu) * lax.rsqrt(var + jnp.float32(EPS))
                  * gam_ref[...] + bet_ref[...])


def _tc_ln(slice_idx, lbl, x, pe, st, gamma, beta):
    nblk = SROWS // BLK
    pe_blocks = SEQ // BLK
    f = pl.pallas_call(
        _tc_body,
        grid=(nblk,),
        in_specs=[
            pl.BlockSpec((BLK, 1), lambda i: (i, 0)),
            pl.BlockSpec((BLK, D), lambda i: (i, 0)),
            pl.BlockSpec((BLK, D),
                         lambda i, s=slice_idx: (
                             (s * (SROWS // BLK) + i) % pe_blocks, 0)),
            pl.BlockSpec((3, D), lambda i: (0, 0)),
            pl.BlockSpec((1, D), lambda i: (0, 0)),
            pl.BlockSpec((1, D), lambda i: (0, 0)),
        ],
        out_specs=pl.BlockSpec((BLK, D), lambda i: (i, 0)),
        out_shape=jax.ShapeDtypeStruct((SROWS, D), jnp.float32),
    )
    return f(lbl, x, pe, st, gamma, beta)


@jax.jit
def _run(seq, seg, token_table, segtab, pe, gamma, beta):
    gathered = [
        _sc_gather(lax.dynamic_slice(seq, (s * SROWS,), (SROWS,)),
                   token_table)
        for s in range(NSLC)
    ]
    outs = [
        _tc_ln(s, lax.dynamic_slice(seg, (s * SROWS, 0), (SROWS, 1)),
               gathered[s], pe, segtab, gamma, beta)
        for s in range(NSLC)
    ]
    if NSLC == 1:
        return outs[0]
    return jnp.concatenate(outs, axis=0)


def kernel(sequence, segment_label, token_table, segment_table, gamma, beta):
    pe = _positional_encoding(SEQ, D)
    out = _run(sequence.reshape(-1), segment_label.reshape(-1, 1),
               token_table, segment_table,
               pe, gamma.reshape(1, D), beta.reshape(1, D))
    return out.reshape(BATCH, SEQ, D)


# speedup vs baseline: 1.2854x; 1.0319x over previous
import functools

import jax
import jax.numpy as jnp
import numpy as np
from jax import lax
from jax.experimental import pallas as pl
from jax.experimental.pallas import tpu as pltpu
from jax.experimental.pallas import tpu_sc as plsc

VOCAB = 100000
D = 768
SEQ = 2048
BATCH = 4
ROWS = BATCH * SEQ
NC, NS, L = 2, 16, 16
NW = NC * NS
NSLC = 1
SROWS = ROWS // NSLC
RPW = SROWS // NW
CHUNK = 64
NCH = RPW // CHUNK
BLK = 512
EPS = 1e-6


def _positional_encoding(seq_len, d_model):
    pos = np.arange(seq_len, dtype=np.float64)[:, None]
    i = np.arange(0, d_model, 2, dtype=np.float64)
    div = np.exp(i * -(np.log(10000.0) / d_model))
    pe = np.zeros((seq_len, d_model), dtype=np.float64)
    pe[:, 0::2] = np.sin(pos * div)
    pe[:, 1::2] = np.cos(pos * div)
    return jnp.asarray(pe, dtype=jnp.float32)


def _sc_gather_body(seq_h, tok_h, out_h, idx_v, b0, b1, gs0, gs1, ws0, ws1):
    wid = lax.axis_index("s") * NC + lax.axis_index("c")
    base0 = wid * RPW
    pltpu.sync_copy(seq_h.at[pl.ds(base0, RPW)], idx_v)

    bufs = (b0, b1)
    gsems = (gs0, gs1)
    wsems = (ws0, ws1)

    def gather(c):
        return pltpu.async_copy(
            tok_h.at[idx_v.at[pl.ds(c * CHUNK, CHUNK)]],
            bufs[c % 2], gsems[c % 2])

    pend_g = gather(0)
    pend_w = [None, None]
    for c in range(NCH):
        pend_g.wait()
        if c + 1 < NCH:
            if pend_w[(c + 1) % 2] is not None:
                pend_w[(c + 1) % 2].wait()
            pend_g = gather(c + 1)
        pend_w[c % 2] = pltpu.async_copy(
            bufs[c % 2], out_h.at[pl.ds(base0 + c * CHUNK, CHUNK)],
            wsems[c % 2])
    for w in pend_w:
        if w is not None:
            w.wait()


def _sc_gather(seq_slice, token_table):
    mesh = plsc.VectorSubcoreMesh(core_axis_name="c", subcore_axis_name="s",
                                  num_cores=NC, num_subcores=NS)
    f = pl.kernel(
        _sc_gather_body,
        out_type=jax.ShapeDtypeStruct((SROWS, D), jnp.float32),
        mesh=mesh,
        scratch_types=[
            pltpu.VMEM((RPW,), jnp.int32),
            pltpu.VMEM((CHUNK, D), jnp.float32),
            pltpu.VMEM((CHUNK, D), jnp.float32),
            pltpu.SemaphoreType.DMA,
            pltpu.SemaphoreType.DMA,
            pltpu.SemaphoreType.DMA,
            pltpu.SemaphoreType.DMA,
        ],
    )
    return f(seq_slice, token_table)


def _tc_body(lbl_ref, x_ref, pe_ref, st_ref, gam_ref, bet_ref, o_ref):
    x = x_ref[...] + pe_ref[...]
    lbl = lbl_ref[...]
    for k in range(3):
        x = x + jnp.where(lbl == k, st_ref[k:k + 1, :],
                          jnp.float32(0.0))
    mu = jnp.mean(x, axis=-1, keepdims=True)
    var = jnp.mean((x - mu) * (x - mu), axis=-1, keepdims=True)
    o_ref[...] = ((x - mu) * lax.rsqrt(var + jnp.float32(EPS))
                  * gam_ref[...] + bet_ref[...])


def _tc_ln(slice_idx, lbl, x, pe, st, gamma, beta):
    npos = SEQ // BLK

    def row_blk(i, b):
        return b * npos + i

    f = pl.pallas_call(
        _tc_body,
        grid=(npos, BATCH),
        in_specs=[
            pl.BlockSpec((BLK, 1), lambda i, b: (row_blk(i, b), 0)),
            pl.BlockSpec((BLK, D), lambda i, b: (row_blk(i, b), 0)),
            pl.BlockSpec((BLK, D), lambda i, b: (i, 0)),
            pl.BlockSpec((3, D), lambda i, b: (0, 0)),
            pl.BlockSpec((1, D), lambda i, b: (0, 0)),
            pl.BlockSpec((1, D), lambda i, b: (0, 0)),
        ],
        out_specs=pl.BlockSpec((BLK, D), lambda i, b: (row_blk(i, b), 0)),
        out_shape=jax.ShapeDtypeStruct((SROWS, D), jnp.float32),
    )
    return f(lbl, x, pe, st, gamma, beta)


@jax.jit
def _run(seq, seg, token_table, segtab, pe, gamma, beta):
    gathered = [
        _sc_gather(lax.dynamic_slice(seq, (s * SROWS,), (SROWS,)),
                   token_table)
        for s in range(NSLC)
    ]
    outs = [
        _tc_ln(s, lax.dynamic_slice(seg, (s * SROWS, 0), (SROWS, 1)),
               gathered[s], pe, segtab, gamma, beta)
        for s in range(NSLC)
    ]
    if NSLC == 1:
        return outs[0]
    return jnp.concatenate(outs, axis=0)


def kernel(sequence, segment_label, token_table, segment_table, gamma, beta):
    pe = _positional_encoding(SEQ, D)
    out = _run(sequence.reshape(-1), segment_label.reshape(-1, 1),
               token_table, segment_table,
               pe, gamma.reshape(1, D), beta.reshape(1, D))
    return out.reshape(BATCH, SEQ, D)


# speedup vs baseline: 1.3525x; 1.0521x over previous
import functools

import jax
import jax.numpy as jnp
import numpy as np
from jax import lax
from jax.experimental import pallas as pl
from jax.experimental.pallas import tpu as pltpu
from jax.experimental.pallas import tpu_sc as plsc

VOCAB = 100000
D = 768
SEQ = 2048
BATCH = 4
ROWS = BATCH * SEQ
NC, NS, L = 2, 16, 16
NW = NC * NS
NSLC = 1
SROWS = ROWS // NSLC
RPW = SROWS // NW
CHUNK = 64
NCH = RPW // CHUNK
BLK = 1024
EPS = 1e-6


def _positional_encoding(seq_len, d_model):
    pos = np.arange(seq_len, dtype=np.float64)[:, None]
    i = np.arange(0, d_model, 2, dtype=np.float64)
    div = np.exp(i * -(np.log(10000.0) / d_model))
    pe = np.zeros((seq_len, d_model), dtype=np.float64)
    pe[:, 0::2] = np.sin(pos * div)
    pe[:, 1::2] = np.cos(pos * div)
    return jnp.asarray(pe, dtype=jnp.float32)


def _sc_gather_body(seq_h, tok_h, out_h, idx_v, b0, b1, gs0, gs1, ws0, ws1):
    wid = lax.axis_index("s") * NC + lax.axis_index("c")
    base0 = wid * RPW
    pltpu.sync_copy(seq_h.at[pl.ds(base0, RPW)], idx_v)

    bufs = (b0, b1)
    gsems = (gs0, gs1)
    wsems = (ws0, ws1)

    def gather(c):
        return pltpu.async_copy(
            tok_h.at[idx_v.at[pl.ds(c * CHUNK, CHUNK)]],
            bufs[c % 2], gsems[c % 2])

    pend_g = gather(0)
    pend_w = [None, None]
    for c in range(NCH):
        pend_g.wait()
        if c + 1 < NCH:
            if pend_w[(c + 1) % 2] is not None:
                pend_w[(c + 1) % 2].wait()
            pend_g = gather(c + 1)
        pend_w[c % 2] = pltpu.async_copy(
            bufs[c % 2], out_h.at[pl.ds(base0 + c * CHUNK, CHUNK)],
            wsems[c % 2])
    for w in pend_w:
        if w is not None:
            w.wait()


def _sc_gather(seq_slice, token_table):
    mesh = plsc.VectorSubcoreMesh(core_axis_name="c", subcore_axis_name="s",
                                  num_cores=NC, num_subcores=NS)
    f = pl.kernel(
        _sc_gather_body,
        out_type=jax.ShapeDtypeStruct((SROWS, D), jnp.float32),
        mesh=mesh,
        scratch_types=[
            pltpu.VMEM((RPW,), jnp.int32),
            pltpu.VMEM((CHUNK, D), jnp.float32),
            pltpu.VMEM((CHUNK, D), jnp.float32),
            pltpu.SemaphoreType.DMA,
            pltpu.SemaphoreType.DMA,
            pltpu.SemaphoreType.DMA,
            pltpu.SemaphoreType.DMA,
        ],
    )
    return f(seq_slice, token_table)


def _tc_body(lbl_ref, x_ref, pe_ref, st_ref, gam_ref, bet_ref, o_ref):
    x = x_ref[...] + pe_ref[...]
    lbl = lbl_ref[...]
    for k in range(3):
        x = x + jnp.where(lbl == k, st_ref[k:k + 1, :],
                          jnp.float32(0.0))
    mu = jnp.mean(x, axis=-1, keepdims=True)
    var = jnp.mean((x - mu) * (x - mu), axis=-1, keepdims=True)
    o_ref[...] = ((x - mu) * lax.rsqrt(var + jnp.float32(EPS))
                  * gam_ref[...] + bet_ref[...])


def _tc_ln(slice_idx, lbl, x, pe, st, gamma, beta):
    npos = SEQ // BLK

    def row_blk(i, b):
        return b * npos + i

    f = pl.pallas_call(
        _tc_body,
        grid=(npos, BATCH),
        in_specs=[
            pl.BlockSpec((BLK, 1), lambda i, b: (row_blk(i, b), 0)),
            pl.BlockSpec((BLK, D), lambda i, b: (row_blk(i, b), 0)),
            pl.BlockSpec((BLK, D), lambda i, b: (i, 0)),
            pl.BlockSpec((3, D), lambda i, b: (0, 0)),
            pl.BlockSpec((1, D), lambda i, b: (0, 0)),
            pl.BlockSpec((1, D), lambda i, b: (0, 0)),
        ],
        out_specs=pl.BlockSpec((BLK, D), lambda i, b: (row_blk(i, b), 0)),
        out_shape=jax.ShapeDtypeStruct((SROWS, D), jnp.float32),
    )
    return f(lbl, x, pe, st, gamma, beta)


@jax.jit
def _run(seq, seg, token_table, segtab, pe, gamma, beta):
    gathered = [
        _sc_gather(lax.dynamic_slice(seq, (s * SROWS,), (SROWS,)),
                   token_table)
        for s in range(NSLC)
    ]
    outs = [
        _tc_ln(s, lax.dynamic_slice(seg, (s * SROWS, 0), (SROWS, 1)),
               gathered[s], pe, segtab, gamma, beta)
        for s in range(NSLC)
    ]
    if NSLC == 1:
        return outs[0]
    return jnp.concatenate(outs, axis=0)


def kernel(sequence, segment_label, token_table, segment_table, gamma, beta):
    pe = _positional_encoding(SEQ, D)
    out = _run(sequence.reshape(-1), segment_label.reshape(-1, 1),
               token_table, segment_table,
               pe, gamma.reshape(1, D), beta.reshape(1, D))
    return out.reshape(BATCH, SEQ, D)


# speedup vs baseline: 1.3986x; 1.0341x over previous
import functools

import jax
import jax.numpy as jnp
import numpy as np
from jax import lax
from jax.experimental import pallas as pl
from jax.experimental.pallas import tpu as pltpu
from jax.experimental.pallas import tpu_sc as plsc

VOCAB = 100000
D = 768
SEQ = 2048
BATCH = 4
ROWS = BATCH * SEQ
NC, NS, L = 2, 16, 16
NW = NC * NS
NSLC = 1
SROWS = ROWS // NSLC
RPW = SROWS // NW
CHUNK = 64
NCH = RPW // CHUNK
BLK = 2048
EPS = 1e-6


def _positional_encoding(seq_len, d_model):
    pos = np.arange(seq_len, dtype=np.float64)[:, None]
    i = np.arange(0, d_model, 2, dtype=np.float64)
    div = np.exp(i * -(np.log(10000.0) / d_model))
    pe = np.zeros((seq_len, d_model), dtype=np.float64)
    pe[:, 0::2] = np.sin(pos * div)
    pe[:, 1::2] = np.cos(pos * div)
    return jnp.asarray(pe, dtype=jnp.float32)


def _sc_gather_body(seq_h, tok_h, out_h, idx_v, b0, b1, gs0, gs1, ws0, ws1):
    wid = lax.axis_index("s") * NC + lax.axis_index("c")
    base0 = wid * RPW
    pltpu.sync_copy(seq_h.at[pl.ds(base0, RPW)], idx_v)

    bufs = (b0, b1)
    gsems = (gs0, gs1)
    wsems = (ws0, ws1)

    def gather(c):
        return pltpu.async_copy(
            tok_h.at[idx_v.at[pl.ds(c * CHUNK, CHUNK)]],
            bufs[c % 2], gsems[c % 2])

    pend_g = gather(0)
    pend_w = [None, None]
    for c in range(NCH):
        pend_g.wait()
        if c + 1 < NCH:
            if pend_w[(c + 1) % 2] is not None:
                pend_w[(c + 1) % 2].wait()
            pend_g = gather(c + 1)
        pend_w[c % 2] = pltpu.async_copy(
            bufs[c % 2], out_h.at[pl.ds(base0 + c * CHUNK, CHUNK)],
            wsems[c % 2])
    for w in pend_w:
        if w is not None:
            w.wait()


def _sc_gather(seq_slice, token_table):
    mesh = plsc.VectorSubcoreMesh(core_axis_name="c", subcore_axis_name="s",
                                  num_cores=NC, num_subcores=NS)
    f = pl.kernel(
        _sc_gather_body,
        out_type=jax.ShapeDtypeStruct((SROWS, D), jnp.float32),
        mesh=mesh,
        scratch_types=[
            pltpu.VMEM((RPW,), jnp.int32),
            pltpu.VMEM((CHUNK, D), jnp.float32),
            pltpu.VMEM((CHUNK, D), jnp.float32),
            pltpu.SemaphoreType.DMA,
            pltpu.SemaphoreType.DMA,
            pltpu.SemaphoreType.DMA,
            pltpu.SemaphoreType.DMA,
        ],
    )
    return f(seq_slice, token_table)


def _tc_body(lbl_ref, x_ref, pe_ref, st_ref, gam_ref, bet_ref, o_ref):
    x = x_ref[...] + pe_ref[...]
    lbl = lbl_ref[...]
    for k in range(3):
        x = x + jnp.where(lbl == k, st_ref[k:k + 1, :],
                          jnp.float32(0.0))
    mu = jnp.mean(x, axis=-1, keepdims=True)
    var = jnp.mean((x - mu) * (x - mu), axis=-1, keepdims=True)
    o_ref[...] = ((x - mu) * lax.rsqrt(var + jnp.float32(EPS))
                  * gam_ref[...] + bet_ref[...])


def _tc_ln(slice_idx, lbl, x, pe, st, gamma, beta):
    npos = SEQ // BLK

    def row_blk(i, b):
        return b * npos + i

    f = pl.pallas_call(
        _tc_body,
        grid=(npos, BATCH),
        in_specs=[
            pl.BlockSpec((BLK, 1), lambda i, b: (row_blk(i, b), 0)),
            pl.BlockSpec((BLK, D), lambda i, b: (row_blk(i, b), 0)),
            pl.BlockSpec((BLK, D), lambda i, b: (i, 0)),
            pl.BlockSpec((3, D), lambda i, b: (0, 0)),
            pl.BlockSpec((1, D), lambda i, b: (0, 0)),
            pl.BlockSpec((1, D), lambda i, b: (0, 0)),
        ],
        out_specs=pl.BlockSpec((BLK, D), lambda i, b: (row_blk(i, b), 0)),
        out_shape=jax.ShapeDtypeStruct((SROWS, D), jnp.float32),
    )
    return f(lbl, x, pe, st, gamma, beta)


@jax.jit
def _run(seq, seg, token_table, segtab, pe, gamma, beta):
    gathered = [
        _sc_gather(lax.dynamic_slice(seq, (s * SROWS,), (SROWS,)),
                   token_table)
        for s in range(NSLC)
    ]
    outs = [
        _tc_ln(s, lax.dynamic_slice(seg, (s * SROWS, 0), (SROWS, 1)),
               gathered[s], pe, segtab, gamma, beta)
        for s in range(NSLC)
    ]
    if NSLC == 1:
        return outs[0]
    return jnp.concatenate(outs, axis=0)


def kernel(sequence, segment_label, token_table, segment_table, gamma, beta):
    pe = _positional_encoding(SEQ, D)
    out = _run(sequence.reshape(-1), segment_label.reshape(-1, 1),
               token_table, segment_table,
               pe, gamma.reshape(1, D), beta.reshape(1, D))
    return out.reshape(BATCH, SEQ, D)
